# baseline (speedup 1.0000x reference)
import functools

import jax
import jax.numpy as jnp
from jax import lax
from jax.experimental import pallas as pl
from jax.experimental.pallas import tpu as pltpu
from jax.experimental.pallas import tpu_sc as plsc

N_ATOM = 10000
E_ATOM = 320000
HID = 128
NGRAPH = 64
NLAYERS = 2

_BR = 2000
_NW = 32
_EPW = E_ATOM // _NW
_B = 40
_NBLK = _EPW // _B
_NPAD = 10240
_RPT = _NPAD // 16


def _mm_body(x_ref, w_ref, b_ref, out_ref):
    out_ref[...] = (
        jnp.dot(x_ref[...], w_ref[...], preferred_element_type=jnp.float32)
        + b_ref[...]
    )


def _mm(x, w, b):
    r, k = x.shape
    f = w.shape[1]
    return pl.pallas_call(
        _mm_body,
        grid=(r // _BR,),
        in_specs=[
            pl.BlockSpec((_BR, k), lambda i: (i, 0)),
            pl.BlockSpec((k, f), lambda i: (0, 0)),
            pl.BlockSpec((1, f), lambda i: (0, 0)),
        ],
        out_specs=pl.BlockSpec((_BR, f), lambda i: (i, 0)),
        out_shape=jax.ShapeDtypeStruct((r, f), jnp.float32),
    )(x, w, b[None, :])


def _add3_body(a_ref, b_ref, c_ref, out_ref):
    out_ref[...] = a_ref[...] + b_ref[...] + c_ref[...]


def _add3(a, b, c):
    r, f = a.shape
    return pl.pallas_call(
        _add3_body,
        grid=(r // _BR,),
        in_specs=[pl.BlockSpec((_BR, f), lambda i: (i, 0))] * 3,
        out_specs=pl.BlockSpec((_BR, f), lambda i: (i, 0)),
        out_shape=jax.ShapeDtypeStruct((r, f), jnp.float32),
    )(a, b, c)


def _pool_body(h_ref, batch_ref, sums_ref, cnt_ref):
    i = pl.program_id(0)

    @pl.when(i == 0)
    def _():
        sums_ref[...] = jnp.zeros_like(sums_ref)
        cnt_ref[...] = jnp.zeros_like(cnt_ref)

    b = batch_ref[0]
    ids = lax.broadcasted_iota(jnp.int32, (NGRAPH, _BR), 0)
    oh = (b == ids).astype(jnp.float32)
    sums_ref[...] += jnp.dot(oh, h_ref[...],
                             preferred_element_type=jnp.float32,
                             precision=lax.Precision.HIGHEST)
    cnt_ref[...] += jnp.sum(oh, axis=1, keepdims=True)


def _pool(h, batch3):
    return pl.pallas_call(
        _pool_body,
        grid=(N_ATOM // _BR,),
        in_specs=[
            pl.BlockSpec((_BR, HID), lambda i: (i, 0)),
            pl.BlockSpec((1, 1, _BR), lambda i: (i, 0, 0)),
        ],
        out_specs=[
            pl.BlockSpec((NGRAPH, HID), lambda i: (0, 0)),
            pl.BlockSpec((NGRAPH, 1), lambda i: (0, 0)),
        ],
        out_shape=[
            jax.ShapeDtypeStruct((NGRAPH, HID), jnp.float32),
            jax.ShapeDtypeStruct((NGRAPH, 1), jnp.float32),
        ],
    )(h, batch3)


def _mlp_body(sums_ref, cnt_ref, w1_ref, b1_ref, w2_ref, b2_ref, out_ref):
    pooled = sums_ref[...] / jnp.maximum(cnt_ref[...], 1.0)
    hid = jnp.maximum(pooled @ w1_ref[...] + b1_ref[...], 0.0)
    out_ref[...] = hid @ w2_ref[...] + b2_ref[...]


def _final_mlp(sums, cnt, w1, b1, w2, b2):
    return pl.pallas_call(
        _mlp_body,
        out_shape=jax.ShapeDtypeStruct((NGRAPH, w2.shape[1]), jnp.float32),
    )(sums, cnt, w1, b1[None, :], w2, b2[None, :])


_CB = 25
_NCHUNK = _NBLK // _CB

_GB = 80
_GNBLK = _EPW // _GB
_GNCH = 5
_SB = 80
_SNBLK = _EPW // _SB
_SNCH = 5


def _gather_sc(xn, xej, xg, xei, src4, dst4):
    mesh = plsc.VectorSubcoreMesh(core_axis_name="c", subcore_axis_name="s")
    out1 = jax.ShapeDtypeStruct((E_ATOM, HID), jnp.float32)

    @functools.partial(
        pl.kernel,
        mesh=mesh,
        out_type=(out1, out1, out1, out1),
        scratch_types=[
            pltpu.VMEM((32, _GB), jnp.int32),
            pltpu.VMEM((32, _GB), jnp.int32),
            pltpu.VMEM((_GB, HID), jnp.float32),
            pltpu.VMEM((_GB, HID), jnp.float32),
            pltpu.VMEM((_GB, HID), jnp.float32),
            pltpu.VMEM((_GB, HID), jnp.float32),
        ] + [pltpu.SemaphoreType.DMA] * 8,
    )
    def k(xn_hbm, xej_hbm, xg_hbm, xei_hbm, s4_hbm, d4_hbm,
          gn_out, gj_out, gg_out, gi_out, sic, dic, nv, jv, gv, iv,
          g0, g1, g2, g3, w0, w1, w2, w3):
        cid = lax.axis_index("c")
        sid = lax.axis_index("s")
        wid = sid * 2 + cid

        def chunk(c, carry):
            crow = wid * _GNBLK + c * _CB
            pltpu.sync_copy(s4_hbm.at[wid, c], sic)
            pltpu.sync_copy(d4_hbm.at[wid, c], dic)

            def block(b, carry2):
                base = (crow + b) * _GB
                si = sic.at[b]
                di = dic.at[b]

                @pl.when((c > 0) | (b > 0))
                def _():
                    pltpu.make_async_copy(
                        nv, gn_out.at[pl.ds(base, _GB)], w0).wait()
                    pltpu.make_async_copy(
                        jv, gj_out.at[pl.ds(base, _GB)], w1).wait()
                    pltpu.make_async_copy(
                        gv, gg_out.at[pl.ds(base, _GB)], w2).wait()
                    pltpu.make_async_copy(
                        iv, gi_out.at[pl.ds(base, _GB)], w3).wait()

                cp0 = pltpu.async_copy(xn_hbm.at[si], nv, g0)
                cp1 = pltpu.async_copy(xej_hbm.at[si], jv, g1)
                cp2 = pltpu.async_copy(xg_hbm.at[di], gv, g2)
                cp3 = pltpu.async_copy(xei_hbm.at[di], iv, g3)
                cp0.wait()
                cp1.wait()
                cp2.wait()
                cp3.wait()
                pltpu.async_copy(nv, gn_out.at[pl.ds(base, _GB)], w0)
                pltpu.async_copy(jv, gj_out.at[pl.ds(base, _GB)], w1)
                pltpu.async_copy(gv, gg_out.at[pl.ds(base, _GB)], w2)
                pltpu.async_copy(iv, gi_out.at[pl.ds(base, _GB)], w3)
                return carry2

            lax.fori_loop(0, _CB, block, 0)
            return carry

        lax.fori_loop(0, _GNCH, chunk, 0)
        last = (wid + 1) * _EPW - _GB
        pltpu.make_async_copy(nv, gn_out.at[pl.ds(last, _GB)], w0).wait()
        pltpu.make_async_copy(jv, gj_out.at[pl.ds(last, _GB)], w1).wait()
        pltpu.make_async_copy(gv, gg_out.at[pl.ds(last, _GB)], w2).wait()
        pltpu.make_async_copy(iv, gi_out.at[pl.ds(last, _GB)], w3).wait()

    return k(xn, xej, xg, xei, src4, dst4)


def _scatter_sc(msg, dst5, zeros):
    mesh = plsc.VectorSubcoreMesh(core_axis_name="c", subcore_axis_name="s")

    @functools.partial(
        pl.kernel,
        mesh=mesh,
        out_type=jax.ShapeDtypeStruct((2, _NPAD, HID), jnp.float32),
        scratch_types=[
            pltpu.VMEM((32, _SB), jnp.int32),
            pltpu.VMEM((_SB, HID), jnp.float32),
            pltpu.VMEM_SHARED((_NPAD, HID), jnp.float32),
            pltpu.SemaphoreType.DMA,
        ],
    )
    def k(msg_hbm, d5_hbm, z_hbm, p_out, dic, mv, acc, sems):
        cid = lax.axis_index("c")
        sid = lax.axis_index("s")
        wid = sid * 2 + cid

        pltpu.sync_copy(z_hbm.at[pl.ds(sid * _RPT, _RPT)],
                        acc.at[pl.ds(sid * _RPT, _RPT)])
        plsc.subcore_barrier()

        def chunk(c, carry):
            crow = wid * _SNBLK + c * _CB

            @pl.when(c > 0)
            def _():
                pltpu.make_async_copy(mv, acc.at[dic.at[_CB - 1]],
                                      sems).wait()

            pltpu.sync_copy(d5_hbm.at[wid, c], dic)

            def block(b, carry2):
                base = (crow + b) * _SB
                di = dic.at[b]

                @pl.when(b > 0)
                def _():
                    pltpu.make_async_copy(mv, acc.at[di], sems).wait()

                pltpu.sync_copy(msg_hbm.at[pl.ds(base, _SB)], mv)
                pltpu.async_copy(mv, acc.at[di], sems, add=True)
                return carry2

            lax.fori_loop(0, _CB, block, 0)
            return carry

        lax.fori_loop(0, _SNCH, chunk, 0)
        pltpu.make_async_copy(mv, acc.at[dic.at[_CB - 1]], sems).wait()
        plsc.subcore_barrier()
        pltpu.sync_copy(acc.at[pl.ds(sid * _RPT, _RPT)],
                        p_out.at[cid, pl.ds(sid * _RPT, _RPT)])

    return k(msg, dst5, zeros)


_EB = 2000


def _edge_tc_body(gn_ref, gj_ref, gg_ref, gi_ref, e_ref,
                  nw_ref, gw_ref, ew_ref, nb_ref, gb_ref, eb_ref,
                  msg_ref, en_ref):
    ev = e_ref[...]
    un = jnp.dot(ev, nw_ref[...], preferred_element_type=jnp.float32)
    ug = jnp.dot(ev, gw_ref[...], preferred_element_type=jnp.float32)
    ue = jnp.dot(ev, ew_ref[...], preferred_element_type=jnp.float32)
    gate = jax.nn.sigmoid(gg_ref[...] + ug + gb_ref[...])
    msg_ref[...] = gate * (gn_ref[...] + un + nb_ref[...])
    en_ref[...] = gj_ref[...] + gi_ref[...] + ue + eb_ref[...] + ev


def _edge_tc(gn, gj, gg, gi, e, nw, gw, ew, nb, gb, eb):
    out1 = jax.ShapeDtypeStruct((E_ATOM, HID), jnp.float32)
    blk = pl.BlockSpec((_EB, HID), lambda i: (i, 0))
    wspec = pl.BlockSpec((HID, HID), lambda i: (0, 0))
    bspec = pl.BlockSpec((1, HID), lambda i: (0, 0))
    return pl.pallas_call(
        _edge_tc_body,
        grid=(E_ATOM // _EB,),
        in_specs=[blk, blk, blk, blk, blk, wspec, wspec, wspec,
                  bspec, bspec, bspec],
        out_specs=[blk, blk],
        out_shape=[out1, out1],
    )(gn, gj, gg, gi, e, nw, gw, ew, nb[None, :], gb[None, :], eb[None, :])


def kernel(atom_x, atom_edge_index, atom_edge_attr, atom_batch, line_x,
           line_edge_index, line_edge_attr, node_embed_W, node_embed_b,
           edge_embed_W, edge_embed_b, line_edge_embed_W, line_edge_embed_b,
           atom_node_W, atom_node_b, atom_edgemlp_W, atom_edgemlp_b,
           atom_gate_W, atom_gate_b, line_node_W, line_node_b,
           line_edgemlp_W, line_edgemlp_b, line_gate_W, line_gate_b,
           mlp_W1, mlp_b1, mlp_W2, mlp_b2):
    def _idx4(v, nch, bb):
        v4 = v.reshape(_NW, nch, _CB, bb)
        return jnp.pad(v4, ((0, 0), (0, 0), (0, 32 - _CB), (0, 0)))

    src4 = _idx4(atom_edge_index[0], _GNCH, _GB)
    dst4 = _idx4(atom_edge_index[1], _GNCH, _GB)
    dst5 = _idx4(atom_edge_index[1], _SNCH, _SB)
    zeros = jnp.zeros((_NPAD, HID), jnp.float32)
    batch3 = atom_batch.reshape(N_ATOM // _BR, 1, _BR).astype(jnp.int32)

    h = _mm(atom_x, node_embed_W, node_embed_b)
    e = _mm(atom_edge_attr, edge_embed_W, edge_embed_b)

    zb = jnp.zeros((HID,), jnp.float32)
    for k in range(NLAYERS):
        nW, nb = atom_node_W[k], atom_node_b[k]
        eW, eb = atom_edgemlp_W[k], atom_edgemlp_b[k]
        gW, gb = atom_gate_W[k], atom_gate_b[k]

        xn = _mm(h, nW[:HID], zb)
        xej = _mm(h, eW[:HID], zb)
        xg = _mm(h, gW[:HID], zb)
        xei = _mm(h, eW[HID:2 * HID], zb)
        gn, gj, gg, gi = _gather_sc(xn, xej, xg, xei, src4, dst4)
        msg, e = _edge_tc(gn, gj, gg, gi, e,
                          nW[HID:], gW[HID:], eW[2 * HID:], nb, gb, eb)
        p = _scatter_sc(msg, dst5, zeros)
        h = _add3(h, p[0, :N_ATOM], p[1, :N_ATOM])

    sums, cnt = _pool(h, batch3)
    return _final_mlp(sums, cnt, mlp_W1, mlp_b1, mlp_W2, mlp_b2)
